# Initial kernel scaffold; baseline (speedup 1.0000x reference)
import functools

import jax
import jax.numpy as jnp
from jax import lax
from jax.experimental import pallas as pl
from jax.experimental.pallas import tpu as pltpu
from jax.experimental.pallas import tpu_sc as plsc

N = 10000
E = 320000
D = 128
H = 16
C = 40
K = 3

NC = 2
NS = 16
NW = NC * NS
CHUNK = 128
CPT = -(-E // (NW * CHUNK))
EPT = CPT * CHUNK
EPAD = NW * EPT
NPAD = NS * (-(-(N + 8) // NS))
ROWS_INIT = NPAD // NS
ROWS_OUT = N // NS

_mesh = plsc.VectorSubcoreMesh(core_axis_name="c", subcore_axis_name="s")


@functools.partial(
    pl.kernel,
    out_type=jax.ShapeDtypeStruct((NC, N, H), jnp.float32),
    mesh=_mesh,
    scratch_types=[
        pltpu.VMEM((CHUNK,), jnp.int32),
        pltpu.VMEM((CHUNK,), jnp.int32),
        pltpu.VMEM((CHUNK, H), jnp.float32),
        pltpu.VMEM_SHARED((NPAD, H), jnp.float32),
        pltpu.SemaphoreType.DMA,
    ],
)
def _sc_prop(u_hbm, src_hbm, dst_hbm, zeros_hbm, out_hbm,
             src_v, dst_v, rows_v, acc, sem):
    cid = lax.axis_index("c")
    sid = lax.axis_index("s")
    wid = sid * NC + cid
    base = wid * EPT

    pltpu.sync_copy(zeros_hbm.at[pl.ds(sid * ROWS_INIT, ROWS_INIT)],
                    acc.at[pl.ds(sid * ROWS_INIT, ROWS_INIT)])
    plsc.subcore_barrier()

    @pl.loop(0, CPT)
    def _(j):
        off = base + j * CHUNK
        pltpu.sync_copy(src_hbm.at[pl.ds(off, CHUNK)], src_v)
        pltpu.sync_copy(dst_hbm.at[pl.ds(off, CHUNK)], dst_v)
        pltpu.async_copy(u_hbm.at[src_v], rows_v, sem).wait()
        pltpu.sync_copy(rows_v, acc.at[dst_v], add=True)

    plsc.subcore_barrier()
    pltpu.sync_copy(acc.at[pl.ds(sid * ROWS_OUT, ROWS_OUT)],
                    out_hbm.at[cid, pl.ds(sid * ROWS_OUT, ROWS_OUT)])


@functools.partial(
    pl.kernel,
    out_type=jax.ShapeDtypeStruct((NC, N, H), jnp.float32),
    mesh=_mesh,
    scratch_types=[
        pltpu.VMEM((CHUNK,), jnp.int32),
        pltpu.VMEM((CHUNK, H), jnp.float32),
        pltpu.VMEM_SHARED((NPAD, H), jnp.float32),
    ],
)
def _sc_deg(dst_hbm, ones_hbm, zeros_hbm, out_hbm, dst_v, ones_v, acc):
    cid = lax.axis_index("c")
    sid = lax.axis_index("s")
    wid = sid * NC + cid
    base = wid * EPT

    pltpu.sync_copy(zeros_hbm.at[pl.ds(sid * ROWS_INIT, ROWS_INIT)],
                    acc.at[pl.ds(sid * ROWS_INIT, ROWS_INIT)])
    pltpu.sync_copy(ones_hbm, ones_v)
    plsc.subcore_barrier()

    @pl.loop(0, CPT)
    def _(j):
        pltpu.sync_copy(dst_hbm.at[pl.ds(base + j * CHUNK, CHUNK)], dst_v)
        pltpu.sync_copy(ones_v, acc.at[dst_v], add=True)

    plsc.subcore_barrier()
    pltpu.sync_copy(acc.at[pl.ds(sid * ROWS_OUT, ROWS_OUT)],
                    out_hbm.at[cid, pl.ds(sid * ROWS_OUT, ROWS_OUT)])


BR = 1000
GRID = N // BR

_b16 = lambda: pl.BlockSpec((BR, H), lambda i: (i, 0))
_b2x16 = lambda: pl.BlockSpec((NC, BR, H), lambda i: (0, i, 0))


def _tc_pre_body(x_ref, w1_ref, degp_ref, u3_ref, dz2_ref, dz1_ref, z0_ref,
                 d2_ref, dinv_ref):
    deg = degp_ref[0] + degp_ref[1]
    dinv = jnp.where(deg > 0, lax.rsqrt(deg), 0.0)
    x = x_ref[...]
    z = [jnp.dot(x, w1_ref[k], preferred_element_type=jnp.float32)
         for k in range(K + 1)]
    u3_ref[...] = dinv * z[3]
    dz2_ref[...] = dinv * z[2]
    dz1_ref[...] = dinv * z[1]
    z0_ref[...] = z[0]
    d2_ref[...] = dinv * dinv
    dinv_ref[...] = dinv


_tc_pre = pl.pallas_call(
    _tc_pre_body,
    grid=(GRID,),
    in_specs=[
        pl.BlockSpec((BR, D), lambda i: (i, 0)),
        pl.BlockSpec((K + 1, D, H), lambda i: (0, 0, 0)),
        _b2x16(),
    ],
    out_specs=[_b16()] * 6,
    out_shape=[jax.ShapeDtypeStruct((N, H), jnp.float32)] * 6,
)


def _tc_comb_body(sp_ref, d2_ref, dz_ref, u_ref):
    u_ref[...] = d2_ref[...] * (sp_ref[0] + sp_ref[1]) + dz_ref[...]


_tc_comb = pl.pallas_call(
    _tc_comb_body,
    grid=(GRID,),
    in_specs=[_b2x16(), _b16(), _b16()],
    out_specs=_b16(),
    out_shape=jax.ShapeDtypeStruct((N, H), jnp.float32),
)


def _tc_relu_body(sp_ref, dinv_ref, z0_ref, b1_ref, x1_ref, u0_ref):
    dinv = dinv_ref[...]
    r = dinv * (sp_ref[0] + sp_ref[1]) + z0_ref[...] + b1_ref[...]
    x1 = jnp.maximum(r, 0.0)
    x1_ref[...] = x1
    u0_ref[...] = dinv * x1


_tc_relu = pl.pallas_call(
    _tc_relu_body,
    grid=(GRID,),
    in_specs=[_b2x16(), _b16(), _b16(),
              pl.BlockSpec((1, H), lambda i: (0, 0))],
    out_specs=[_b16(), _b16()],
    out_shape=[jax.ShapeDtypeStruct((N, H), jnp.float32)] * 2,
)


def _tc_final_body(x1_ref, t1_ref, t2_ref, t3_ref, dinv_ref, w2_ref, b2_ref,
                   out_ref):
    dinv = dinv_ref[...]
    dot = lambda a, w: jnp.dot(a, w, preferred_element_type=jnp.float32)
    logits = dot(x1_ref[...], w2_ref[0])
    for k, t_ref in enumerate((t1_ref, t2_ref, t3_ref)):
        logits += dot(dinv * (t_ref[0] + t_ref[1]), w2_ref[k + 1])
    logits += b2_ref[...]
    m = jnp.max(logits, axis=1, keepdims=True)
    shifted = logits - m
    lse = jnp.log(jnp.sum(jnp.exp(shifted), axis=1, keepdims=True))
    out_ref[...] = shifted - lse


_tc_final = pl.pallas_call(
    _tc_final_body,
    grid=(GRID,),
    in_specs=[_b16(), _b2x16(), _b2x16(), _b2x16(), _b16(),
              pl.BlockSpec((K + 1, H, C), lambda i: (0, 0, 0)),
              pl.BlockSpec((1, C), lambda i: (0, 0))],
    out_specs=pl.BlockSpec((BR, C), lambda i: (i, 0)),
    out_shape=jax.ShapeDtypeStruct((N, C), jnp.float32),
)


def kernel(train_data, train_edge_index, training, W1, b1, W2, b2):
    src = train_edge_index[0].astype(jnp.int32)
    dst = train_edge_index[1].astype(jnp.int32)
    pad = EPAD - E
    src_p = jnp.concatenate([src, jnp.zeros((pad,), jnp.int32)])
    dst_p = jnp.concatenate([dst, jnp.full((pad,), N, jnp.int32)])

    zeros = jnp.zeros((NPAD, H), jnp.float32)
    ones = jnp.ones((CHUNK, H), jnp.float32)

    degp = _sc_deg(dst_p, ones, zeros)
    u3, dz2, dz1, z0, d2, dinv = _tc_pre(train_data, W1, degp)

    zero16 = jnp.zeros((N, H), jnp.float32)
    s3 = _sc_prop(u3, src_p, dst_p, zeros)
    u2 = _tc_comb(s3, d2, dz2)
    s2 = _sc_prop(u2, src_p, dst_p, zeros)
    u1 = _tc_comb(s2, d2, dz1)
    s1 = _sc_prop(u1, src_p, dst_p, zeros)
    x1, u0 = _tc_relu(s1, dinv, z0, b1.reshape(1, H))

    t1 = _sc_prop(u0, src_p, dst_p, zeros)
    v1 = _tc_comb(t1, d2, zero16)
    t2 = _sc_prop(v1, src_p, dst_p, zeros)
    v2 = _tc_comb(t2, d2, zero16)
    t3 = _sc_prop(v2, src_p, dst_p, zeros)

    return _tc_final(x1, t1, t2, t3, dinv, W2, b2.reshape(1, C))


# speedup vs baseline: 11.4526x; 11.4526x over previous
import functools

import jax
import jax.numpy as jnp
from jax import lax
from jax.experimental import pallas as pl
from jax.experimental.pallas import tpu as pltpu
from jax.experimental.pallas import tpu_sc as plsc

N = 10000
E = 320000
D = 128
H = 16
C = 40
K = 3

NC = 2
NS = 16
NW = NC * NS
CHUNK = 128
CPT = -(-E // (NW * CHUNK))
EPT = CPT * CHUNK
EPAD = NW * EPT
ROWS = 8 * (-(-N // (8 * NS)))
NPAD = NS * ROWS

_mesh = plsc.VectorSubcoreMesh(core_axis_name="c", subcore_axis_name="s")


@functools.partial(
    pl.kernel,
    out_type=jax.ShapeDtypeStruct((NC, NPAD, H), jnp.float32),
    mesh=_mesh,
    compiler_params=pltpu.CompilerParams(use_tc_tiling_on_sc=False),
    scratch_types=[
        pltpu.VMEM((CHUNK,), jnp.int32),
        pltpu.VMEM((CHUNK,), jnp.int32),
        pltpu.VMEM((CHUNK, H), jnp.float32),
        pltpu.VMEM_SHARED((NPAD, H), jnp.float32),
        pltpu.SemaphoreType.DMA,
    ],
)
def _sc_prop(u_hbm, src_hbm, dst_hbm, zeros_hbm, out_hbm,
             src_v, dst_v, rows_v, acc, sem):
    cid = lax.axis_index("c")
    sid = lax.axis_index("s")
    wid = sid * NC + cid
    base = wid * EPT
    rows = pl.ds(sid * ROWS, ROWS)

    pltpu.sync_copy(zeros_hbm.at[rows], acc.at[rows])
    plsc.subcore_barrier()

    @pl.loop(0, CPT)
    def _(j):
        off = base + j * CHUNK
        pltpu.sync_copy(src_hbm.at[pl.ds(off, CHUNK)], src_v)
        pltpu.sync_copy(dst_hbm.at[pl.ds(off, CHUNK)], dst_v)
        pltpu.async_copy(u_hbm.at[src_v], rows_v, sem).wait()
        pltpu.async_copy(rows_v, acc.at[dst_v], sem, add=True).wait()

    plsc.subcore_barrier()
    pltpu.sync_copy(acc.at[rows], out_hbm.at[cid, rows])


@functools.partial(
    pl.kernel,
    out_type=jax.ShapeDtypeStruct((NC, NPAD, H), jnp.float32),
    mesh=_mesh,
    compiler_params=pltpu.CompilerParams(use_tc_tiling_on_sc=False),
    scratch_types=[
        pltpu.VMEM((CHUNK,), jnp.int32),
        pltpu.VMEM((CHUNK, H), jnp.float32),
        pltpu.VMEM_SHARED((NPAD, H), jnp.float32),
        pltpu.SemaphoreType.DMA,
    ],
)
def _sc_deg(dst_hbm, ones_hbm, zeros_hbm, out_hbm, dst_v, ones_v, acc, sem):
    cid = lax.axis_index("c")
    sid = lax.axis_index("s")
    wid = sid * NC + cid
    base = wid * EPT
    rows = pl.ds(sid * ROWS, ROWS)

    pltpu.sync_copy(zeros_hbm.at[rows], acc.at[rows])
    pltpu.sync_copy(ones_hbm, ones_v)
    plsc.subcore_barrier()

    @pl.loop(0, CPT)
    def _(j):
        pltpu.sync_copy(dst_hbm.at[pl.ds(base + j * CHUNK, CHUNK)], dst_v)
        pltpu.async_copy(ones_v, acc.at[dst_v], sem, add=True).wait()

    plsc.subcore_barrier()
    pltpu.sync_copy(acc.at[rows], out_hbm.at[cid, rows])


BR = ROWS
GRID = NPAD // BR

_b16 = lambda: pl.BlockSpec((BR, H), lambda i: (i, 0))
_b2x16 = lambda: pl.BlockSpec((NC, BR, H), lambda i: (0, i, 0))


def _tc_pre_body(x_ref, w1_ref, degp_ref, u3_ref, dz2_ref, dz1_ref, z0_ref,
                 d2_ref, dinv_ref):
    deg = degp_ref[0] + degp_ref[1]
    dinv = jnp.where(deg > 0, lax.rsqrt(deg), 0.0)
    x = x_ref[...]
    z = [jnp.dot(x, w1_ref[k], preferred_element_type=jnp.float32)
         for k in range(K + 1)]
    u3_ref[...] = dinv * z[3]
    dz2_ref[...] = dinv * z[2]
    dz1_ref[...] = dinv * z[1]
    z0_ref[...] = z[0]
    d2_ref[...] = dinv * dinv
    dinv_ref[...] = dinv


_tc_pre = pl.pallas_call(
    _tc_pre_body,
    grid=(GRID,),
    in_specs=[
        pl.BlockSpec((BR, D), lambda i: (i, 0)),
        pl.BlockSpec((K + 1, D, H), lambda i: (0, 0, 0)),
        _b2x16(),
    ],
    out_specs=[_b16()] * 6,
    out_shape=[jax.ShapeDtypeStruct((NPAD, H), jnp.float32)] * 6,
)


def _tc_comb_body(sp_ref, d2_ref, dz_ref, u_ref):
    u_ref[...] = d2_ref[...] * (sp_ref[0] + sp_ref[1]) + dz_ref[...]


_tc_comb = pl.pallas_call(
    _tc_comb_body,
    grid=(GRID,),
    in_specs=[_b2x16(), _b16(), _b16()],
    out_specs=_b16(),
    out_shape=jax.ShapeDtypeStruct((NPAD, H), jnp.float32),
)


def _tc_relu_body(sp_ref, dinv_ref, z0_ref, b1_ref, x1_ref, u0_ref):
    dinv = dinv_ref[...]
    r = dinv * (sp_ref[0] + sp_ref[1]) + z0_ref[...] + b1_ref[...]
    x1 = jnp.maximum(r, 0.0)
    x1_ref[...] = x1
    u0_ref[...] = dinv * x1


_tc_relu = pl.pallas_call(
    _tc_relu_body,
    grid=(GRID,),
    in_specs=[_b2x16(), _b16(), _b16(),
              pl.BlockSpec((1, H), lambda i: (0, 0))],
    out_specs=[_b16(), _b16()],
    out_shape=[jax.ShapeDtypeStruct((NPAD, H), jnp.float32)] * 2,
)


BRF = 1000
GRIDF = N // BRF

_f16 = lambda: pl.BlockSpec((BRF, H), lambda i: (i, 0))
_f2x16 = lambda: pl.BlockSpec((NC, BRF, H), lambda i: (0, i, 0))


def _tc_final_body(x1_ref, t1_ref, t2_ref, t3_ref, dinv_ref, w2_ref, b2_ref,
                   out_ref):
    dinv = dinv_ref[...]
    dot = lambda a, w: jnp.dot(a, w, preferred_element_type=jnp.float32)
    logits = dot(x1_ref[...], w2_ref[0])
    for k, t_ref in enumerate((t1_ref, t2_ref, t3_ref)):
        logits += dot(dinv * (t_ref[0] + t_ref[1]), w2_ref[k + 1])
    logits += b2_ref[...]
    m = jnp.max(logits, axis=1, keepdims=True)
    shifted = logits - m
    lse = jnp.log(jnp.sum(jnp.exp(shifted), axis=1, keepdims=True))
    out_ref[...] = shifted - lse


_tc_final = pl.pallas_call(
    _tc_final_body,
    grid=(GRIDF,),
    in_specs=[_f16(), _f2x16(), _f2x16(), _f2x16(), _f16(),
              pl.BlockSpec((K + 1, H, C), lambda i: (0, 0, 0)),
              pl.BlockSpec((1, C), lambda i: (0, 0))],
    out_specs=pl.BlockSpec((BRF, C), lambda i: (i, 0)),
    out_shape=jax.ShapeDtypeStruct((N, C), jnp.float32),
)


def kernel(train_data, train_edge_index, training, W1, b1, W2, b2):
    src = train_edge_index[0].astype(jnp.int32)
    dst = train_edge_index[1].astype(jnp.int32)
    pad = EPAD - E
    src_p = jnp.concatenate([src, jnp.zeros((pad,), jnp.int32)])
    dst_p = jnp.concatenate([dst, jnp.full((pad,), N, jnp.int32)])
    x_p = jnp.pad(train_data, ((0, NPAD - N), (0, 0)))

    zeros = jnp.zeros((NPAD, H), jnp.float32)
    ones = jnp.ones((CHUNK, H), jnp.float32)

    degp = _sc_deg(dst_p, ones, zeros)
    u3, dz2, dz1, z0, d2, dinv = _tc_pre(x_p, W1, degp)

    s3 = _sc_prop(u3, src_p, dst_p, zeros)
    u2 = _tc_comb(s3, d2, dz2)
    s2 = _sc_prop(u2, src_p, dst_p, zeros)
    u1 = _tc_comb(s2, d2, dz1)
    s1 = _sc_prop(u1, src_p, dst_p, zeros)
    x1, u0 = _tc_relu(s1, dinv, z0, b1.reshape(1, H))

    t1 = _sc_prop(u0, src_p, dst_p, zeros)
    v1 = _tc_comb(t1, d2, zeros)
    t2 = _sc_prop(v1, src_p, dst_p, zeros)
    v2 = _tc_comb(t2, d2, zeros)
    t3 = _sc_prop(v2, src_p, dst_p, zeros)

    return _tc_final(x1, t1, t2, t3, dinv, W2, b2.reshape(1, C))


# speedup vs baseline: 19.8245x; 1.7310x over previous
import functools

import jax
import jax.numpy as jnp
from jax import lax
from jax.experimental import pallas as pl
from jax.experimental.pallas import tpu as pltpu
from jax.experimental.pallas import tpu_sc as plsc

N = 10000
E = 320000
D = 128
H = 16
C = 40
K = 3

NC = 2
NS = 16
NW = NC * NS
CHUNK = 128
NBUF = 8
CPT = NBUF * (-(-E // (NW * CHUNK * NBUF)))
EPT = CPT * CHUNK
EPAD = NW * EPT
ROWS = 8 * (-(-N // (8 * NS)))
NPAD = NS * ROWS

_mesh = plsc.VectorSubcoreMesh(core_axis_name="c", subcore_axis_name="s")


@functools.partial(
    pl.kernel,
    out_type=jax.ShapeDtypeStruct((NC, NPAD, H), jnp.float32),
    mesh=_mesh,
    compiler_params=pltpu.CompilerParams(use_tc_tiling_on_sc=False),
    scratch_types=[
        pltpu.VMEM((EPT,), jnp.int32),
        pltpu.VMEM((EPT,), jnp.int32),
        pltpu.VMEM((NBUF, CHUNK, H), jnp.float32),
        pltpu.VMEM_SHARED((NPAD, H), jnp.float32),
        pltpu.SemaphoreType.DMA((NBUF,)),
        pltpu.SemaphoreType.DMA((NBUF,)),
    ],
)
def _sc_prop(u_hbm, src_hbm, dst_hbm, zeros_hbm, out_hbm,
             src_all, dst_all, rows_v, acc, gsem, ssem):
    cid = lax.axis_index("c")
    sid = lax.axis_index("s")
    wid = sid * NC + cid
    base = wid * EPT
    rows = pl.ds(sid * ROWS, ROWS)

    pltpu.sync_copy(src_hbm.at[pl.ds(base, EPT)], src_all)
    pltpu.sync_copy(dst_hbm.at[pl.ds(base, EPT)], dst_all)
    pltpu.sync_copy(zeros_hbm.at[rows], acc.at[rows])
    plsc.subcore_barrier()

    @pl.loop(0, CPT // NBUF)
    def _(g):
        gds = []
        for b in range(NBUF):
            ix = src_all.at[pl.ds((g * NBUF + b) * CHUNK, CHUNK)]
            gds.append(pltpu.async_copy(u_hbm.at[ix], rows_v.at[b],
                                        gsem.at[b]))
        sds = []
        for b in range(NBUF):
            gds[b].wait()
            ox = dst_all.at[pl.ds((g * NBUF + b) * CHUNK, CHUNK)]
            sds.append(pltpu.async_copy(rows_v.at[b], acc.at[ox],
                                        ssem.at[b], add=True))
        for sd in sds:
            sd.wait()

    plsc.subcore_barrier()
    pltpu.sync_copy(acc.at[rows], out_hbm.at[cid, rows])


@functools.partial(
    pl.kernel,
    out_type=jax.ShapeDtypeStruct((NC, NPAD, H), jnp.float32),
    mesh=_mesh,
    compiler_params=pltpu.CompilerParams(use_tc_tiling_on_sc=False),
    scratch_types=[
        pltpu.VMEM((EPT,), jnp.int32),
        pltpu.VMEM((CHUNK, H), jnp.float32),
        pltpu.VMEM_SHARED((NPAD, H), jnp.float32),
        pltpu.SemaphoreType.DMA((NBUF,)),
    ],
)
def _sc_deg(dst_hbm, ones_hbm, zeros_hbm, out_hbm, dst_all, ones_v, acc, ssem):
    cid = lax.axis_index("c")
    sid = lax.axis_index("s")
    wid = sid * NC + cid
    base = wid * EPT
    rows = pl.ds(sid * ROWS, ROWS)

    pltpu.sync_copy(dst_hbm.at[pl.ds(base, EPT)], dst_all)
    pltpu.sync_copy(zeros_hbm.at[rows], acc.at[rows])
    pltpu.sync_copy(ones_hbm, ones_v)
    plsc.subcore_barrier()

    @pl.loop(0, CPT // NBUF)
    def _(g):
        sds = []
        for b in range(NBUF):
            ox = dst_all.at[pl.ds((g * NBUF + b) * CHUNK, CHUNK)]
            sds.append(pltpu.async_copy(ones_v, acc.at[ox],
                                        ssem.at[b], add=True))
        for sd in sds:
            sd.wait()

    plsc.subcore_barrier()
    pltpu.sync_copy(acc.at[rows], out_hbm.at[cid, rows])


BR = ROWS
GRID = NPAD // BR

_b16 = lambda: pl.BlockSpec((BR, H), lambda i: (i, 0))
_b2x16 = lambda: pl.BlockSpec((NC, BR, H), lambda i: (0, i, 0))


def _tc_pre_body(x_ref, w1_ref, degp_ref, u3_ref, dz2_ref, dz1_ref, z0_ref,
                 d2_ref, dinv_ref):
    deg = degp_ref[0] + degp_ref[1]
    dinv = jnp.where(deg > 0, lax.rsqrt(deg), 0.0)
    x = x_ref[...]
    z = [jnp.dot(x, w1_ref[k], preferred_element_type=jnp.float32)
         for k in range(K + 1)]
    u3_ref[...] = dinv * z[3]
    dz2_ref[...] = dinv * z[2]
    dz1_ref[...] = dinv * z[1]
    z0_ref[...] = z[0]
    d2_ref[...] = dinv * dinv
    dinv_ref[...] = dinv


_tc_pre = pl.pallas_call(
    _tc_pre_body,
    grid=(GRID,),
    in_specs=[
        pl.BlockSpec((BR, D), lambda i: (i, 0)),
        pl.BlockSpec((K + 1, D, H), lambda i: (0, 0, 0)),
        _b2x16(),
    ],
    out_specs=[_b16()] * 6,
    out_shape=[jax.ShapeDtypeStruct((NPAD, H), jnp.float32)] * 6,
)


def _tc_comb_body(sp_ref, d2_ref, dz_ref, u_ref):
    u_ref[...] = d2_ref[...] * (sp_ref[0] + sp_ref[1]) + dz_ref[...]


_tc_comb = pl.pallas_call(
    _tc_comb_body,
    grid=(GRID,),
    in_specs=[_b2x16(), _b16(), _b16()],
    out_specs=_b16(),
    out_shape=jax.ShapeDtypeStruct((NPAD, H), jnp.float32),
)


def _tc_relu_body(sp_ref, dinv_ref, z0_ref, b1_ref, x1_ref, u0_ref):
    dinv = dinv_ref[...]
    r = dinv * (sp_ref[0] + sp_ref[1]) + z0_ref[...] + b1_ref[...]
    x1 = jnp.maximum(r, 0.0)
    x1_ref[...] = x1
    u0_ref[...] = dinv * x1


_tc_relu = pl.pallas_call(
    _tc_relu_body,
    grid=(GRID,),
    in_specs=[_b2x16(), _b16(), _b16(),
              pl.BlockSpec((1, H), lambda i: (0, 0))],
    out_specs=[_b16(), _b16()],
    out_shape=[jax.ShapeDtypeStruct((NPAD, H), jnp.float32)] * 2,
)


BRF = 1000
GRIDF = N // BRF

_f16 = lambda: pl.BlockSpec((BRF, H), lambda i: (i, 0))
_f2x16 = lambda: pl.BlockSpec((NC, BRF, H), lambda i: (0, i, 0))


def _tc_final_body(x1_ref, t1_ref, t2_ref, t3_ref, dinv_ref, w2_ref, b2_ref,
                   out_ref):
    dinv = dinv_ref[...]
    dot = lambda a, w: jnp.dot(a, w, preferred_element_type=jnp.float32)
    logits = dot(x1_ref[...], w2_ref[0])
    for k, t_ref in enumerate((t1_ref, t2_ref, t3_ref)):
        logits += dot(dinv * (t_ref[0] + t_ref[1]), w2_ref[k + 1])
    logits += b2_ref[...]
    m = jnp.max(logits, axis=1, keepdims=True)
    shifted = logits - m
    lse = jnp.log(jnp.sum(jnp.exp(shifted), axis=1, keepdims=True))
    out_ref[...] = shifted - lse


_tc_final = pl.pallas_call(
    _tc_final_body,
    grid=(GRIDF,),
    in_specs=[_f16(), _f2x16(), _f2x16(), _f2x16(), _f16(),
              pl.BlockSpec((K + 1, H, C), lambda i: (0, 0, 0)),
              pl.BlockSpec((1, C), lambda i: (0, 0))],
    out_specs=pl.BlockSpec((BRF, C), lambda i: (i, 0)),
    out_shape=jax.ShapeDtypeStruct((N, C), jnp.float32),
)


def kernel(train_data, train_edge_index, training, W1, b1, W2, b2):
    src = train_edge_index[0].astype(jnp.int32)
    dst = train_edge_index[1].astype(jnp.int32)
    pad = EPAD - E
    src_p = jnp.concatenate([src, jnp.zeros((pad,), jnp.int32)])
    dst_p = jnp.concatenate([dst, jnp.full((pad,), N, jnp.int32)])
    x_p = jnp.pad(train_data, ((0, NPAD - N), (0, 0)))

    zeros = jnp.zeros((NPAD, H), jnp.float32)
    ones = jnp.ones((CHUNK, H), jnp.float32)

    degp = _sc_deg(dst_p, ones, zeros)
    u3, dz2, dz1, z0, d2, dinv = _tc_pre(x_p, W1, degp)

    s3 = _sc_prop(u3, src_p, dst_p, zeros)
    u2 = _tc_comb(s3, d2, dz2)
    s2 = _sc_prop(u2, src_p, dst_p, zeros)
    u1 = _tc_comb(s2, d2, dz1)
    s1 = _sc_prop(u1, src_p, dst_p, zeros)
    x1, u0 = _tc_relu(s1, dinv, z0, b1.reshape(1, H))

    t1 = _sc_prop(u0, src_p, dst_p, zeros)
    v1 = _tc_comb(t1, d2, zeros)
    t2 = _sc_prop(v1, src_p, dst_p, zeros)
    v2 = _tc_comb(t2, d2, zeros)
    t3 = _sc_prop(v2, src_p, dst_p, zeros)

    return _tc_final(x1, t1, t2, t3, dinv, W2, b2.reshape(1, C))


# speedup vs baseline: 39.2690x; 1.9808x over previous
import functools

import jax
import jax.numpy as jnp
from jax import lax
from jax.experimental import pallas as pl
from jax.experimental.pallas import tpu as pltpu
from jax.experimental.pallas import tpu_sc as plsc

N = 10000
E = 320000
D = 128
H = 16
C = 40
K = 3

NC = 2
NS = 16
NW = NC * NS
CHUNK = 512
NBUF = 4
CPT = NBUF * (-(-E // (NW * CHUNK * NBUF)))
EPT = CPT * CHUNK
EPAD = NW * EPT
ROWS = 8 * (-(-N // (8 * NS)))
NPAD = NS * ROWS

_mesh = plsc.VectorSubcoreMesh(core_axis_name="c", subcore_axis_name="s")
_params = pltpu.CompilerParams(use_tc_tiling_on_sc=False)


def _propagate(u_s, acc, src_all, dst_all, rows_v, gsem, ssem):
    @pl.loop(0, CPT // NBUF)
    def _(g):
        gds = []
        for b in range(NBUF):
            ix = src_all.at[pl.ds((g * NBUF + b) * CHUNK, CHUNK)]
            gds.append(pltpu.async_copy(u_s.at[ix], rows_v.at[b],
                                        gsem.at[b]))
        sds = []
        for b in range(NBUF):
            gds[b].wait()
            ox = dst_all.at[pl.ds((g * NBUF + b) * CHUNK, CHUNK)]
            sds.append(pltpu.async_copy(rows_v.at[b], acc.at[ox],
                                        ssem.at[b], add=True))
        for sd in sds:
            sd.wait()


_SC_SCRATCH = [
    pltpu.VMEM((EPT,), jnp.int32),
    pltpu.VMEM((EPT,), jnp.int32),
    pltpu.VMEM((NBUF, CHUNK, H), jnp.float32),
    pltpu.VMEM_SHARED((NPAD, H), jnp.float32),
    pltpu.VMEM_SHARED((NPAD, H), jnp.float32),
    pltpu.SemaphoreType.DMA((NBUF,)),
    pltpu.SemaphoreType.DMA((NBUF,)),
]
_EW_SCRATCH = [
    pltpu.VMEM((ROWS, H), jnp.float32),
    pltpu.VMEM((ROWS, H), jnp.float32),
    pltpu.VMEM((ROWS, H), jnp.float32),
    pltpu.VMEM((ROWS, H), jnp.float32),
    pltpu.VMEM((ROWS, H), jnp.float32),
]


@functools.partial(
    pl.kernel,
    out_type=jax.ShapeDtypeStruct((NC, NPAD, H), jnp.float32),
    mesh=_mesh,
    compiler_params=_params,
    scratch_types=_SC_SCRATCH,
)
def _sc_prop(u_hbm, src_hbm, dst_hbm, zeros_hbm, out_hbm,
             src_all, dst_all, rows_v, u_s, acc, gsem, ssem):
    cid = lax.axis_index("c")
    sid = lax.axis_index("s")
    base = (sid * NC + cid) * EPT
    rows = pl.ds(sid * ROWS, ROWS)

    pltpu.sync_copy(src_hbm.at[pl.ds(base, EPT)], src_all)
    pltpu.sync_copy(dst_hbm.at[pl.ds(base, EPT)], dst_all)
    pltpu.sync_copy(u_hbm.at[rows], u_s.at[rows])
    pltpu.sync_copy(zeros_hbm.at[rows], acc.at[rows])
    plsc.subcore_barrier()

    _propagate(u_s, acc, src_all, dst_all, rows_v, gsem, ssem)

    plsc.subcore_barrier()
    pltpu.sync_copy(acc.at[rows], out_hbm.at[cid, rows])


@functools.partial(
    pl.kernel,
    out_type=jax.ShapeDtypeStruct((NC, NPAD, H), jnp.float32),
    mesh=_mesh,
    compiler_params=_params,
    scratch_types=_SC_SCRATCH + _EW_SCRATCH,
)
def _sc_prop_lin(p_hbm, m_hbm, a_hbm, src_hbm, dst_hbm, zeros_hbm, out_hbm,
                 src_all, dst_all, rows_v, u_s, acc, gsem, ssem,
                 p0v, p1v, mv, av, uv):
    cid = lax.axis_index("c")
    sid = lax.axis_index("s")
    base = (sid * NC + cid) * EPT
    rows = pl.ds(sid * ROWS, ROWS)

    pltpu.sync_copy(src_hbm.at[pl.ds(base, EPT)], src_all)
    pltpu.sync_copy(dst_hbm.at[pl.ds(base, EPT)], dst_all)
    pltpu.sync_copy(p_hbm.at[0, rows], p0v)
    pltpu.sync_copy(p_hbm.at[1, rows], p1v)
    pltpu.sync_copy(m_hbm.at[rows], mv)
    pltpu.sync_copy(a_hbm.at[rows], av)
    pltpu.sync_copy(zeros_hbm.at[rows], acc.at[rows])

    @pl.loop(0, ROWS)
    def _(r):
        uv[r] = mv[r] * (p0v[r] + p1v[r]) + av[r]

    pltpu.sync_copy(uv, u_s.at[rows])
    plsc.subcore_barrier()

    _propagate(u_s, acc, src_all, dst_all, rows_v, gsem, ssem)

    plsc.subcore_barrier()
    pltpu.sync_copy(acc.at[rows], out_hbm.at[cid, rows])


@functools.partial(
    pl.kernel,
    out_type=(jax.ShapeDtypeStruct((NC, NPAD, H), jnp.float32),
              jax.ShapeDtypeStruct((NPAD, H), jnp.float32)),
    mesh=_mesh,
    compiler_params=_params,
    scratch_types=_SC_SCRATCH + _EW_SCRATCH,
)
def _sc_prop_relu(p_hbm, m_hbm, a_hbm, src_hbm, dst_hbm, zeros_hbm,
                  out_hbm, x1_hbm,
                  src_all, dst_all, rows_v, u_s, acc, gsem, ssem,
                  p0v, p1v, mv, av, uv):
    cid = lax.axis_index("c")
    sid = lax.axis_index("s")
    base = (sid * NC + cid) * EPT
    rows = pl.ds(sid * ROWS, ROWS)

    pltpu.sync_copy(src_hbm.at[pl.ds(base, EPT)], src_all)
    pltpu.sync_copy(dst_hbm.at[pl.ds(base, EPT)], dst_all)
    pltpu.sync_copy(p_hbm.at[0, rows], p0v)
    pltpu.sync_copy(p_hbm.at[1, rows], p1v)
    pltpu.sync_copy(m_hbm.at[rows], mv)
    pltpu.sync_copy(a_hbm.at[rows], av)
    pltpu.sync_copy(zeros_hbm.at[rows], acc.at[rows])

    @pl.loop(0, ROWS)
    def _(r):
        x = jnp.maximum(mv[r] * (p0v[r] + p1v[r]) + av[r], 0.0)
        uv[r] = x
        p0v[r] = mv[r] * x

    pltpu.sync_copy(uv, x1_hbm.at[rows])
    pltpu.sync_copy(p0v, u_s.at[rows])
    plsc.subcore_barrier()

    _propagate(u_s, acc, src_all, dst_all, rows_v, gsem, ssem)

    plsc.subcore_barrier()
    pltpu.sync_copy(acc.at[rows], out_hbm.at[cid, rows])


@functools.partial(
    pl.kernel,
    out_type=jax.ShapeDtypeStruct((NC, NPAD, H), jnp.float32),
    mesh=_mesh,
    compiler_params=_params,
    scratch_types=[
        pltpu.VMEM((EPT,), jnp.int32),
        pltpu.VMEM((CHUNK, H), jnp.float32),
        pltpu.VMEM_SHARED((NPAD, H), jnp.float32),
        pltpu.SemaphoreType.DMA((NBUF,)),
    ],
)
def _sc_deg(dst_hbm, ones_hbm, zeros_hbm, out_hbm, dst_all, ones_v, acc, ssem):
    cid = lax.axis_index("c")
    sid = lax.axis_index("s")
    base = (sid * NC + cid) * EPT
    rows = pl.ds(sid * ROWS, ROWS)

    pltpu.sync_copy(dst_hbm.at[pl.ds(base, EPT)], dst_all)
    pltpu.sync_copy(zeros_hbm.at[rows], acc.at[rows])
    pltpu.sync_copy(ones_hbm, ones_v)
    plsc.subcore_barrier()

    @pl.loop(0, CPT // NBUF)
    def _(g):
        sds = []
        for b in range(NBUF):
            ox = dst_all.at[pl.ds((g * NBUF + b) * CHUNK, CHUNK)]
            sds.append(pltpu.async_copy(ones_v, acc.at[ox],
                                        ssem.at[b], add=True))
        for sd in sds:
            sd.wait()

    plsc.subcore_barrier()
    pltpu.sync_copy(acc.at[rows], out_hbm.at[cid, rows])


BR = ROWS
GRID = NPAD // BR

_b16 = lambda: pl.BlockSpec((BR, H), lambda i: (i, 0))
_b2x16 = lambda: pl.BlockSpec((NC, BR, H), lambda i: (0, i, 0))


def _tc_pre_body(x_ref, w1_ref, b1_ref, degp_ref, u3_ref, dz2_ref, dz1_ref,
                 z0b_ref, d2_ref, dinv_ref):
    deg = degp_ref[0] + degp_ref[1]
    dinv = jnp.where(deg > 0, lax.rsqrt(deg), 0.0)
    x = x_ref[...]
    z = [jnp.dot(x, w1_ref[k], preferred_element_type=jnp.float32)
         for k in range(K + 1)]
    u3_ref[...] = dinv * z[3]
    dz2_ref[...] = dinv * z[2]
    dz1_ref[...] = dinv * z[1]
    z0b_ref[...] = z[0] + b1_ref[...]
    d2_ref[...] = dinv * dinv
    dinv_ref[...] = dinv


_tc_pre = pl.pallas_call(
    _tc_pre_body,
    grid=(GRID,),
    in_specs=[
        pl.BlockSpec((BR, D), lambda i: (i, 0)),
        pl.BlockSpec((K + 1, D, H), lambda i: (0, 0, 0)),
        pl.BlockSpec((1, H), lambda i: (0, 0)),
        _b2x16(),
    ],
    out_specs=[_b16()] * 6,
    out_shape=[jax.ShapeDtypeStruct((NPAD, H), jnp.float32)] * 6,
)


BRF = 1000
GRIDF = N // BRF

_f16 = lambda: pl.BlockSpec((BRF, H), lambda i: (i, 0))
_f2x16 = lambda: pl.BlockSpec((NC, BRF, H), lambda i: (0, i, 0))


def _tc_final_body(x1_ref, t1_ref, t2_ref, t3_ref, dinv_ref, w2_ref, b2_ref,
                   out_ref):
    dinv = dinv_ref[...]
    dot = lambda a, w: jnp.dot(a, w, preferred_element_type=jnp.float32)
    logits = dot(x1_ref[...], w2_ref[0])
    for k, t_ref in enumerate((t1_ref, t2_ref, t3_ref)):
        logits += dot(dinv * (t_ref[0] + t_ref[1]), w2_ref[k + 1])
    logits += b2_ref[...]
    m = jnp.max(logits, axis=1, keepdims=True)
    shifted = logits - m
    lse = jnp.log(jnp.sum(jnp.exp(shifted), axis=1, keepdims=True))
    out_ref[...] = shifted - lse


_tc_final = pl.pallas_call(
    _tc_final_body,
    grid=(GRIDF,),
    in_specs=[_f16(), _f2x16(), _f2x16(), _f2x16(), _f16(),
              pl.BlockSpec((K + 1, H, C), lambda i: (0, 0, 0)),
              pl.BlockSpec((1, C), lambda i: (0, 0))],
    out_specs=pl.BlockSpec((BRF, C), lambda i: (i, 0)),
    out_shape=jax.ShapeDtypeStruct((N, C), jnp.float32),
)


def kernel(train_data, train_edge_index, training, W1, b1, W2, b2):
    src = train_edge_index[0].astype(jnp.int32)
    dst = train_edge_index[1].astype(jnp.int32)
    pad = EPAD - E
    src_p = jnp.concatenate([src, jnp.zeros((pad,), jnp.int32)])
    dst_p = jnp.concatenate([dst, jnp.full((pad,), N, jnp.int32)])
    x_p = jnp.pad(train_data, ((0, NPAD - N), (0, 0)))

    zeros = jnp.zeros((NPAD, H), jnp.float32)
    ones = jnp.ones((CHUNK, H), jnp.float32)

    degp = _sc_deg(dst_p, ones, zeros)
    u3, dz2, dz1, z0b, d2, dinv = _tc_pre(x_p, W1, b1.reshape(1, H), degp)

    s3 = _sc_prop(u3, src_p, dst_p, zeros)
    s2 = _sc_prop_lin(s3, d2, dz2, src_p, dst_p, zeros)
    s1 = _sc_prop_lin(s2, d2, dz1, src_p, dst_p, zeros)
    t1, x1 = _sc_prop_relu(s1, dinv, z0b, src_p, dst_p, zeros)
    t2 = _sc_prop_lin(t1, d2, zeros, src_p, dst_p, zeros)
    t3 = _sc_prop_lin(t2, d2, zeros, src_p, dst_p, zeros)

    return _tc_final(x1, t1, t2, t3, dinv, W2, b2.reshape(1, C))
